# baseline (speedup 1.0000x reference)
import functools

import jax
import jax.numpy as jnp
from jax import lax
from jax.experimental import pallas as pl
from jax.experimental.pallas import tpu as pltpu
from jax.experimental.pallas import tpu_sc as plsc

_NUM_CORES = 2
_NUM_SUBCORES = 16
_NUM_WORKERS = _NUM_CORES * _NUM_SUBCORES


def kernel(inputs, input_table, position_table):
    batch, seqlen = inputs.shape
    vocab, embed = input_table.shape
    chunk = seqlen // _NUM_WORKERS
    rpw = batch * chunk

    mesh = plsc.VectorSubcoreMesh(
        core_axis_name="c", subcore_axis_name="s",
        num_cores=_NUM_CORES, num_subcores=_NUM_SUBCORES)

    @functools.partial(
        pl.kernel,
        out_type=jax.ShapeDtypeStruct((batch, seqlen, embed), jnp.float32),
        mesh=mesh,
        scratch_types=[
            pltpu.VMEM((batch, chunk), jnp.int32),
            pltpu.VMEM((batch, chunk), jnp.int32),
            pltpu.VMEM((rpw, embed), jnp.float32),
            pltpu.VMEM_SHARED((_NUM_SUBCORES * rpw, embed), jnp.float32),
            pltpu.SemaphoreType.DMA((4,)),
            pltpu.SemaphoreType.DMA((4,)),
            pltpu.SemaphoreType.DMA((4,)),
            pltpu.SemaphoreType.DMA((4,)),
            pltpu.SemaphoreType.DMA,
        ],
    )
    def emb_kernel(idx_hbm, tab_hbm, pos_hbm, out_hbm,
                   idx_v, scat_v, rows_v, shared,
                   sem_p, sem_i, sem_g, sem_a, sem_o):
        c = lax.axis_index("c")
        s = lax.axis_index("s")
        wid = s * _NUM_CORES + c
        col = wid * chunk
        base = s * rpw
        pos_src = pos_hbm.at[pl.ds(col, chunk)]
        cp_pos = [
            pltpu.async_copy(
                pos_src, shared.at[pl.ds(base + b * chunk, chunk)],
                sem_p.at[b])
            for b in range(batch)
        ]
        cp_idx = [
            pltpu.async_copy(idx_hbm.at[b, pl.ds(col, chunk)], idx_v.at[b],
                             sem_i.at[b])
            for b in range(batch)
        ]
        lanes = lax.iota(jnp.int32, 16)
        for b in range(batch):
            for k in range(chunk // 16):
                scat_v[b, pl.ds(k * 16, 16)] = lanes + (
                    base + b * chunk + k * 16)
        gathers = []
        for b in range(batch):
            cp_idx[b].wait()
            gathers.append(pltpu.async_copy(
                tab_hbm.at[idx_v.at[b]],
                rows_v.at[pl.ds(b * chunk, chunk)], sem_g.at[b]))
        sadds = []
        for b in range(batch):
            gathers[b].wait()
            cp_pos[b].wait()
            sadds.append(pltpu.async_copy(
                rows_v.at[pl.ds(b * chunk, chunk)],
                shared.at[scat_v.at[b]], sem_a.at[b], add=True))
        outs = []
        for b in range(batch):
            sadds[b].wait()
            outs.append(pltpu.async_copy(
                shared.at[pl.ds(base + b * chunk, chunk)],
                out_hbm.at[b, pl.ds(col, chunk)], sem_o))
        for o in outs:
            o.wait()

    return emb_kernel(inputs, input_table, position_table)
